# baseline (speedup 1.0000x reference)
import functools

import jax
import jax.numpy as jnp
from jax import lax
from jax.experimental import pallas as pl
from jax.experimental.pallas import tpu as pltpu
from jax.experimental.pallas import tpu_sc as plsc

BATCH = 4096
MAXLEN = 200
EMBED = 64

_info = plsc.get_sparse_core_info()
NC = _info.num_cores
NS = _info.num_subcores
LANES = _info.num_lanes
NW = NC * NS
BATCH_PER_W = BATCH // NW
BB = 4
NCHUNKS = BATCH_PER_W // BB
GROUPS = [(off, min(128, MAXLEN - off)) for off in range(0, MAXLEN, 128)]
JVECS = EMBED // LANES


def _tec_body(idx_hbm, table_hbm, pos_hbm, out_hbm,
              pos_v, idx0, idx1, buf0, buf1, sem):
    wid = lax.axis_index("s") * NC + lax.axis_index("c")
    wbase = wid * BATCH_PER_W

    pltpu.sync_copy(pos_hbm, pos_v)

    def load_and_fire(g, idxb, buf):
        pltpu.sync_copy(idx_hbm.at[pl.ds(wbase + g * BB, BB)], idxb)
        for b in range(BB):
            for off, sz in GROUPS:
                pltpu.async_copy(table_hbm.at[idxb.at[b, pl.ds(off, sz)]],
                                 buf.at[b, pl.ds(off, sz)], sem)

    def drain(buf):
        for b in range(BB):
            for off, sz in GROUPS:
                pltpu.make_async_copy(table_hbm.at[pl.ds(0, sz)],
                                      buf.at[b, pl.ds(off, sz)], sem).wait()

    def add_pos(buf):
        def body(r, c):
            for j in range(JVECS):
                p = pos_v[r, pl.ds(j * LANES, LANES)]
                for b in range(BB):
                    plsc.addupdate(buf.at[b, r, pl.ds(j * LANES, LANES)], p)
            return c
        lax.fori_loop(0, MAXLEN, body, 0)

    load_and_fire(0, idx0, buf0)

    def outer(i, c):
        for half in range(2):
            g = 2 * i + half
            idxb, buf = (idx0, buf0) if half == 0 else (idx1, buf1)
            nidx, nbuf = (idx1, buf1) if half == 0 else (idx0, buf0)
            drain(buf)

            @pl.when(g + 1 < NCHUNKS)
            def _():
                load_and_fire(g + 1, nidx, nbuf)

            add_pos(buf)
            pltpu.sync_copy(buf, out_hbm.at[pl.ds(wbase + g * BB, BB)])
        return c

    lax.fori_loop(0, NCHUNKS // 2, outer, 0)


_emb_call = functools.partial(
    pl.kernel,
    out_type=jax.ShapeDtypeStruct((BATCH, MAXLEN, EMBED), jnp.float32),
    mesh=plsc.VectorSubcoreMesh(core_axis_name="c", subcore_axis_name="s"),
    compiler_params=pltpu.CompilerParams(use_tc_tiling_on_sc=False),
    scratch_types=[
        pltpu.VMEM((MAXLEN, EMBED), jnp.float32),
        pltpu.VMEM((BB, MAXLEN), jnp.int32),
        pltpu.VMEM((BB, MAXLEN), jnp.int32),
        pltpu.VMEM((BB, MAXLEN, EMBED), jnp.float32),
        pltpu.VMEM((BB, MAXLEN, EMBED), jnp.float32),
        pltpu.SemaphoreType.DMA,
    ],
)(_tec_body)


def kernel(inputs, token_emb, pos_emb):
    return _emb_call(inputs.astype(jnp.int32), token_emb, pos_emb)


# speedup vs baseline: 1.0022x; 1.0022x over previous
import functools

import jax
import jax.numpy as jnp
from jax import lax
from jax.experimental import pallas as pl
from jax.experimental.pallas import tpu as pltpu
from jax.experimental.pallas import tpu_sc as plsc

BATCH = 4096
MAXLEN = 200
EMBED = 64

_info = plsc.get_sparse_core_info()
NC = _info.num_cores
NS = _info.num_subcores
LANES = _info.num_lanes
NW = NC * NS
BATCH_PER_W = BATCH // NW
BB = 4
NCHUNKS = BATCH_PER_W // BB
GROUPS = [(off, min(128, MAXLEN - off)) for off in range(0, MAXLEN, 128)]
JVECS = EMBED // LANES


def _tec_body(idx_hbm, table_hbm, pos_hbm, out_hbm,
              pos_v, idx0, idx1, buf0, buf1, sem, wsem):
    wid = lax.axis_index("s") * NC + lax.axis_index("c")
    wbase = wid * BATCH_PER_W

    pltpu.sync_copy(pos_hbm, pos_v)

    def load_and_fire(g, idxb, buf):
        pltpu.sync_copy(idx_hbm.at[pl.ds(wbase + g * BB, BB)], idxb)
        for b in range(BB):
            for off, sz in GROUPS:
                pltpu.async_copy(table_hbm.at[idxb.at[b, pl.ds(off, sz)]],
                                 buf.at[b, pl.ds(off, sz)], sem)

    def drain(buf):
        for b in range(BB):
            for off, sz in GROUPS:
                pltpu.make_async_copy(table_hbm.at[pl.ds(0, sz)],
                                      buf.at[b, pl.ds(off, sz)], sem).wait()

    def add_pos(buf):
        def body(r, c):
            for j in range(JVECS):
                p = pos_v[r, pl.ds(j * LANES, LANES)]
                for b in range(BB):
                    plsc.addupdate(buf.at[b, r, pl.ds(j * LANES, LANES)], p)
            return c
        lax.fori_loop(0, MAXLEN, body, 0)

    def wait_wb(buf, g):
        pltpu.make_async_copy(buf, out_hbm.at[pl.ds(wbase + g * BB, BB)],
                              wsem).wait()

    load_and_fire(0, idx0, buf0)

    def outer(i, c):
        for half in range(2):
            g = 2 * i + half
            idxb, buf = (idx0, buf0) if half == 0 else (idx1, buf1)
            nidx, nbuf = (idx1, buf1) if half == 0 else (idx0, buf0)
            drain(buf)

            @pl.when(g + 1 < NCHUNKS)
            def _():
                @pl.when(g >= 1)
                def _():
                    wait_wb(nbuf, g - 1)
                load_and_fire(g + 1, nidx, nbuf)

            add_pos(buf)
            pltpu.async_copy(buf, out_hbm.at[pl.ds(wbase + g * BB, BB)], wsem)
        return c

    lax.fori_loop(0, NCHUNKS // 2, outer, 0)
    wait_wb(buf0, NCHUNKS - 2)
    wait_wb(buf1, NCHUNKS - 1)


_emb_call = functools.partial(
    pl.kernel,
    out_type=jax.ShapeDtypeStruct((BATCH, MAXLEN, EMBED), jnp.float32),
    mesh=plsc.VectorSubcoreMesh(core_axis_name="c", subcore_axis_name="s"),
    compiler_params=pltpu.CompilerParams(use_tc_tiling_on_sc=False),
    scratch_types=[
        pltpu.VMEM((MAXLEN, EMBED), jnp.float32),
        pltpu.VMEM((BB, MAXLEN), jnp.int32),
        pltpu.VMEM((BB, MAXLEN), jnp.int32),
        pltpu.VMEM((BB, MAXLEN, EMBED), jnp.float32),
        pltpu.VMEM((BB, MAXLEN, EMBED), jnp.float32),
        pltpu.SemaphoreType.DMA,
        pltpu.SemaphoreType.DMA,
    ],
)(_tec_body)


def kernel(inputs, token_emb, pos_emb):
    return _emb_call(inputs.astype(jnp.int32), token_emb, pos_emb)
